# baseline (speedup 1.0000x reference)
import functools
import math

import jax
import jax.numpy as jnp
from jax import lax
from jax.experimental import pallas as pl
from jax.experimental.pallas import tpu as pltpu
from jax.experimental.pallas import tpu_sc as plsc

RANK = 96
EMB_DIM = 192
SCALE = 1.0 / math.sqrt(EMB_DIM)

NUM_CORES = 2
NUM_SUBCORES = 16
NUM_WORKERS = NUM_CORES * NUM_SUBCORES
CHUNK = 128


def _sc_gather(table, idx_flat):
    b_total = idx_flat.shape[0]
    per_w = b_total // NUM_WORKERS
    n_chunks = per_w // CHUNK

    mesh = plsc.VectorSubcoreMesh(core_axis_name="c", subcore_axis_name="s")

    @functools.partial(
        pl.kernel,
        out_type=jax.ShapeDtypeStruct((b_total, RANK), jnp.float32),
        mesh=mesh,
        compiler_params=pltpu.CompilerParams(use_tc_tiling_on_sc=False),
        scratch_types=[
            pltpu.VMEM((per_w,), jnp.int32),
            pltpu.VMEM((CHUNK, RANK), jnp.float32),
            pltpu.SemaphoreType.DMA,
        ],
    )
    def gather_kernel(table_hbm, idx_hbm, out_hbm, idx_v, rows_v, sem):
        wid = lax.axis_index("s") * NUM_CORES + lax.axis_index("c")
        base = wid * per_w
        pltpu.sync_copy(idx_hbm.at[pl.ds(base, per_w)], idx_v)

        def body(j, carry):
            off = j * CHUNK
            pltpu.async_copy(
                table_hbm.at[idx_v.at[pl.ds(off, CHUNK)]], rows_v, sem
            ).wait()
            pltpu.sync_copy(rows_v, out_hbm.at[pl.ds(base + off, CHUNK)])
            return carry

        lax.fori_loop(0, n_chunks, body, 0)

    return gather_kernel(table, idx_flat)


def _tc_project(low_flat, proj_w):
    b_total = low_flat.shape[0]
    blk = 2048
    grid = b_total // blk

    def mm_kernel(low_ref, w_ref, out_ref):
        out_ref[...] = lax.dot_general(
            low_ref[...], w_ref[...],
            (((1,), (1,)), ((), ())),
            preferred_element_type=jnp.float32,
        ) * SCALE

    return pl.pallas_call(
        mm_kernel,
        grid=(grid,),
        in_specs=[
            pl.BlockSpec((blk, RANK), lambda i: (i, 0)),
            pl.BlockSpec((EMB_DIM, RANK), lambda i: (0, 0)),
        ],
        out_specs=pl.BlockSpec((blk, EMB_DIM), lambda i: (i, 0)),
        out_shape=jax.ShapeDtypeStruct((b_total, EMB_DIM), jnp.float32),
    )(low_flat, proj_w)


def kernel(x, emb_low, proj_w):
    bsz, seq = x.shape
    idx_flat = x.reshape(-1)
    low = _sc_gather(emb_low, idx_flat)
    out = _tc_project(low, proj_w)
    return out.reshape(bsz, seq, EMB_DIM)


# speedup vs baseline: 3.7528x; 3.7528x over previous
import functools
import math

import jax
import jax.numpy as jnp
from jax import lax
from jax.experimental import pallas as pl
from jax.experimental.pallas import tpu as pltpu
from jax.experimental.pallas import tpu_sc as plsc

RANK = 96
EMB_DIM = 192
PAD = 128
SCALE = 1.0 / math.sqrt(EMB_DIM)

NUM_CORES = 2
NUM_SUBCORES = 16
NUM_WORKERS = NUM_CORES * NUM_SUBCORES
CHUNK = 128


def _tc_repack(table_t, proj_pad):
    vocab = table_t.shape[1]
    blk = 2048
    grid = (vocab + blk - 1) // blk

    def repack_kernel(tt_ref, eye_ref, out_ref):
        out_ref[...] = lax.dot_general(
            tt_ref[...], eye_ref[...],
            (((0,), (0,)), ((), ())),
            preferred_element_type=jnp.float32,
        )

    return pl.pallas_call(
        repack_kernel,
        grid=(grid,),
        in_specs=[
            pl.BlockSpec((RANK, blk), lambda i: (0, i)),
            pl.BlockSpec((RANK, PAD), lambda i: (0, 0)),
        ],
        out_specs=pl.BlockSpec((blk, PAD), lambda i: (i, 0)),
        out_shape=jax.ShapeDtypeStruct((vocab, PAD), jnp.float32),
    )(table_t, proj_pad)


def _sc_gather(table_pad, idx_flat):
    b_total = idx_flat.shape[0]
    per_w = b_total // NUM_WORKERS
    n_chunks = per_w // CHUNK

    mesh = plsc.VectorSubcoreMesh(core_axis_name="c", subcore_axis_name="s")

    @functools.partial(
        pl.kernel,
        out_type=jax.ShapeDtypeStruct((b_total, PAD), jnp.float32),
        mesh=mesh,
        compiler_params=pltpu.CompilerParams(use_tc_tiling_on_sc=False),
        scratch_types=[
            pltpu.VMEM((per_w,), jnp.int32),
            pltpu.VMEM((CHUNK, PAD), jnp.float32),
            pltpu.SemaphoreType.DMA,
        ],
    )
    def gather_kernel(table_hbm, idx_hbm, out_hbm, idx_v, rows_v, sem):
        wid = lax.axis_index("s") * NUM_CORES + lax.axis_index("c")
        base = wid * per_w
        pltpu.sync_copy(idx_hbm.at[pl.ds(base, per_w)], idx_v)

        def body(j, carry):
            off = j * CHUNK
            pltpu.async_copy(
                table_hbm.at[idx_v.at[pl.ds(off, CHUNK)]], rows_v, sem
            ).wait()
            pltpu.sync_copy(rows_v, out_hbm.at[pl.ds(base + off, CHUNK)])
            return carry

        lax.fori_loop(0, n_chunks, body, 0)

    return gather_kernel(table_pad, idx_flat)


def _tc_project(low3d, proj_pad):
    seq, bsz, _ = low3d.shape

    def mm_kernel(low_ref, w_ref, out_ref):
        out_ref[0] = lax.dot_general(
            w_ref[...], low_ref[0],
            (((1,), (1,)), ((), ())),
            preferred_element_type=jnp.float32,
        ) * SCALE

    return pl.pallas_call(
        mm_kernel,
        grid=(seq,),
        in_specs=[
            pl.BlockSpec((1, bsz, PAD), lambda i: (i, 0, 0)),
            pl.BlockSpec((EMB_DIM, PAD), lambda i: (0, 0)),
        ],
        out_specs=pl.BlockSpec((1, EMB_DIM, bsz), lambda i: (i, 0, 0)),
        out_shape=jax.ShapeDtypeStruct((seq, EMB_DIM, bsz), jnp.float32),
    )(low3d, proj_pad)


def kernel(x, emb_low, proj_w):
    bsz, seq = x.shape
    table_t = emb_low.T
    idx_flat = x.T.reshape(-1)
    proj_pad = jnp.pad(proj_w, ((0, 0), (0, PAD - RANK)))
    eye_pad = jnp.pad(jnp.eye(RANK, dtype=jnp.float32), ((0, 0), (0, PAD - RANK)))

    table_pad = _tc_repack(table_t, eye_pad)
    low = _sc_gather(table_pad, idx_flat)
    low3d = low.reshape(seq, bsz, PAD)
    out = _tc_project(low3d, proj_pad)
    return out.transpose(2, 0, 1)


# speedup vs baseline: 4.9615x; 1.3221x over previous
import functools
import math

import jax
import jax.numpy as jnp
from jax import lax
from jax.experimental import pallas as pl
from jax.experimental.pallas import tpu as pltpu
from jax.experimental.pallas import tpu_sc as plsc

RANK = 96
EMB_DIM = 192
PAD = 128
SCALE = 1.0 / math.sqrt(EMB_DIM)

NUM_CORES = 2
NUM_SUBCORES = 16
NUM_WORKERS = NUM_CORES * NUM_SUBCORES
CHUNK = 128


def _tc_repack(table_t, proj_pad):
    vocab = table_t.shape[1]
    blk = 4096
    grid = (vocab + blk - 1) // blk

    def repack_kernel(tt_ref, eye_ref, out_ref):
        out_ref[...] = lax.dot_general(
            tt_ref[...], eye_ref[...],
            (((0,), (0,)), ((), ())),
            preferred_element_type=jnp.float32,
        )

    return pl.pallas_call(
        repack_kernel,
        grid=(grid,),
        in_specs=[
            pl.BlockSpec((RANK, blk), lambda i: (0, i)),
            pl.BlockSpec((RANK, PAD), lambda i: (0, 0)),
        ],
        out_specs=pl.BlockSpec((blk, PAD), lambda i: (i, 0)),
        out_shape=jax.ShapeDtypeStruct((vocab, PAD), jnp.float32),
    )(table_t, proj_pad)


def _sc_gather(table_pad, idx_flat):
    b_total = idx_flat.shape[0]
    per_w = b_total // NUM_WORKERS
    n_chunks = per_w // CHUNK

    mesh = plsc.VectorSubcoreMesh(core_axis_name="c", subcore_axis_name="s")

    @functools.partial(
        pl.kernel,
        out_type=jax.ShapeDtypeStruct((b_total, PAD), jnp.float32),
        mesh=mesh,
        compiler_params=pltpu.CompilerParams(use_tc_tiling_on_sc=False),
        scratch_types=[
            pltpu.VMEM((per_w,), jnp.int32),
            pltpu.VMEM((CHUNK, PAD), jnp.float32),
            pltpu.VMEM((CHUNK, PAD), jnp.float32),
            pltpu.SemaphoreType.DMA,
            pltpu.SemaphoreType.DMA,
        ],
    )
    def gather_kernel(table_hbm, idx_hbm, out_hbm, idx_v, rows0, rows1, sem0, sem1):
        wid = lax.axis_index("s") * NUM_CORES + lax.axis_index("c")
        base = wid * per_w
        pltpu.sync_copy(idx_hbm.at[pl.ds(base, per_w)], idx_v)

        def start(j, buf, sem):
            pltpu.async_copy(
                table_hbm.at[idx_v.at[pl.ds(j * CHUNK, CHUNK)]], buf, sem)

        def drain(buf, sem):
            pltpu.make_async_copy(
                table_hbm.at[pl.ds(0, CHUNK)], buf, sem).wait()

        n_pairs = n_chunks // 2
        start(0, rows0, sem0)

        def pair_body(p, carry):
            j0 = 2 * p
            start(j0 + 1, rows1, sem1)
            drain(rows0, sem0)
            pltpu.sync_copy(rows0, out_hbm.at[pl.ds(base + j0 * CHUNK, CHUNK)])

            @pl.when(p + 1 < n_pairs)
            def _():
                start(j0 + 2, rows0, sem0)

            drain(rows1, sem1)
            pltpu.sync_copy(
                rows1, out_hbm.at[pl.ds(base + (j0 + 1) * CHUNK, CHUNK)])
            return carry

        lax.fori_loop(0, n_pairs, pair_body, 0)

    return gather_kernel(table_pad, idx_flat)


def _tc_project(low3d, proj_pad):
    seq, bsz, _ = low3d.shape

    def mm_kernel(low_ref, w_ref, out_ref):
        out_ref[0] = lax.dot_general(
            w_ref[...], low_ref[0],
            (((1,), (1,)), ((), ())),
            preferred_element_type=jnp.float32,
        ) * SCALE

    return pl.pallas_call(
        mm_kernel,
        grid=(seq,),
        in_specs=[
            pl.BlockSpec((1, bsz, PAD), lambda i: (i, 0, 0)),
            pl.BlockSpec((EMB_DIM, PAD), lambda i: (0, 0)),
        ],
        out_specs=pl.BlockSpec((1, EMB_DIM, bsz), lambda i: (i, 0, 0)),
        out_shape=jax.ShapeDtypeStruct((seq, EMB_DIM, bsz), jnp.float32),
    )(low3d, proj_pad)


def kernel(x, emb_low, proj_w):
    bsz, seq = x.shape
    table_t = emb_low.T
    idx_flat = x.T.reshape(-1)
    proj_pad = jnp.pad(proj_w, ((0, 0), (0, PAD - RANK)))
    eye_pad = jnp.pad(jnp.eye(RANK, dtype=jnp.float32), ((0, 0), (0, PAD - RANK)))

    table_pad = _tc_repack(table_t, eye_pad)
    low = _sc_gather(table_pad, idx_flat)
    low3d = low.reshape(seq, bsz, PAD)
    out = _tc_project(low3d, proj_pad)
    return out.transpose(2, 0, 1)


# speedup vs baseline: 5.8259x; 1.1742x over previous
import functools
import math

import jax
import jax.numpy as jnp
from jax import lax
from jax.experimental import pallas as pl
from jax.experimental.pallas import tpu as pltpu
from jax.experimental.pallas import tpu_sc as plsc

RANK = 96
EMB_DIM = 192
PAD = 128
SCALE = 1.0 / math.sqrt(EMB_DIM)

NUM_CORES = 2
NUM_SUBCORES = 16
NUM_WORKERS = NUM_CORES * NUM_SUBCORES
CHUNK = 128


def _tc_repack(table_t, proj_pad):
    vocab = table_t.shape[1]
    blk = 8192
    grid = (vocab + blk - 1) // blk

    def repack_kernel(tt_ref, eye_ref, out_ref):
        out_ref[...] = lax.dot_general(
            tt_ref[...], eye_ref[...],
            (((0,), (0,)), ((), ())),
            preferred_element_type=jnp.float32,
        )

    return pl.pallas_call(
        repack_kernel,
        grid=(grid,),
        in_specs=[
            pl.BlockSpec((RANK, blk), lambda i: (0, i)),
            pl.BlockSpec((RANK, PAD), lambda i: (0, 0)),
        ],
        out_specs=pl.BlockSpec((blk, PAD), lambda i: (i, 0)),
        out_shape=jax.ShapeDtypeStruct((vocab, PAD), jnp.float32),
    )(table_t, proj_pad)


def _sc_gather(table_pad, idx_flat):
    b_total = idx_flat.shape[0]
    per_w = b_total // NUM_WORKERS
    n_chunks = per_w // CHUNK

    mesh = plsc.VectorSubcoreMesh(core_axis_name="c", subcore_axis_name="s")

    @functools.partial(
        pl.kernel,
        out_type=jax.ShapeDtypeStruct((b_total, PAD), jnp.float32),
        mesh=mesh,
        compiler_params=pltpu.CompilerParams(use_tc_tiling_on_sc=False),
        scratch_types=[
            pltpu.VMEM((per_w,), jnp.int32),
            pltpu.VMEM((CHUNK, PAD), jnp.float32),
            pltpu.VMEM((CHUNK, PAD), jnp.float32),
            pltpu.SemaphoreType.DMA,
            pltpu.SemaphoreType.DMA,
        ],
    )
    def gather_kernel(table_hbm, idx_hbm, out_hbm, idx_v, rows0, rows1, sem0, sem1):
        wid = lax.axis_index("s") * NUM_CORES + lax.axis_index("c")
        base = wid * per_w
        pltpu.sync_copy(idx_hbm.at[pl.ds(base, per_w)], idx_v)

        def start(j, buf, sem):
            pltpu.async_copy(
                table_hbm.at[idx_v.at[pl.ds(j * CHUNK, CHUNK)]], buf, sem)

        def drain(buf, sem):
            pltpu.make_async_copy(
                table_hbm.at[pl.ds(0, CHUNK)], buf, sem).wait()

        n_pairs = n_chunks // 2
        start(0, rows0, sem0)

        def pair_body(p, carry):
            j0 = 2 * p
            start(j0 + 1, rows1, sem1)
            drain(rows0, sem0)
            pltpu.sync_copy(rows0, out_hbm.at[pl.ds(base + j0 * CHUNK, CHUNK)])

            @pl.when(p + 1 < n_pairs)
            def _():
                start(j0 + 2, rows0, sem0)

            drain(rows1, sem1)
            pltpu.sync_copy(
                rows1, out_hbm.at[pl.ds(base + (j0 + 1) * CHUNK, CHUNK)])
            return carry

        lax.fori_loop(0, n_pairs, pair_body, 0)

    return gather_kernel(table_pad, idx_flat)


def _tc_project(low3d, proj_pad):
    seq, bsz, _ = low3d.shape

    lpb = 2

    def mm_kernel(low_ref, w_ref, out_ref):
        for r in range(lpb):
            out_ref[r] = lax.dot_general(
                w_ref[...], low_ref[r],
                (((1,), (1,)), ((), ())),
                preferred_element_type=jnp.float32,
            ) * SCALE

    return pl.pallas_call(
        mm_kernel,
        grid=(seq // lpb,),
        in_specs=[
            pl.BlockSpec((lpb, bsz, PAD), lambda i: (i, 0, 0)),
            pl.BlockSpec((EMB_DIM, PAD), lambda i: (0, 0)),
        ],
        out_specs=pl.BlockSpec((lpb, EMB_DIM, bsz), lambda i: (i, 0, 0)),
        out_shape=jax.ShapeDtypeStruct((seq, EMB_DIM, bsz), jnp.float32),
    )(low3d, proj_pad)


def kernel(x, emb_low, proj_w):
    bsz, seq = x.shape
    table_t = emb_low.T
    idx_flat = x.T.reshape(-1)
    proj_pad = jnp.pad(proj_w, ((0, 0), (0, PAD - RANK)))
    eye_pad = jnp.pad(jnp.eye(RANK, dtype=jnp.float32), ((0, 0), (0, PAD - RANK)))

    table_pad = _tc_repack(table_t, eye_pad)
    low = _sc_gather(table_pad, idx_flat)
    low3d = low.reshape(seq, bsz, PAD)
    out = _tc_project(low3d, proj_pad)
    return out.transpose(2, 0, 1)


# speedup vs baseline: 5.9658x; 1.0240x over previous
import functools
import math

import jax
import jax.numpy as jnp
from jax import lax
from jax.experimental import pallas as pl
from jax.experimental.pallas import tpu as pltpu
from jax.experimental.pallas import tpu_sc as plsc

RANK = 96
EMB_DIM = 192
PAD = 128
SCALE = 1.0 / math.sqrt(EMB_DIM)

NUM_CORES = 2
NUM_SUBCORES = 16
NUM_WORKERS = NUM_CORES * NUM_SUBCORES
CHUNK = 128


def _tc_repack(table_t, proj_pad):
    vocab = table_t.shape[1]
    blk = 16384
    grid = (vocab + blk - 1) // blk

    def repack_kernel(tt_ref, eye_ref, out_ref):
        out_ref[...] = lax.dot_general(
            tt_ref[...], eye_ref[...],
            (((0,), (0,)), ((), ())),
            preferred_element_type=jnp.float32,
        )

    return pl.pallas_call(
        repack_kernel,
        grid=(grid,),
        in_specs=[
            pl.BlockSpec((RANK, blk), lambda i: (0, i)),
            pl.BlockSpec((RANK, PAD), lambda i: (0, 0)),
        ],
        out_specs=pl.BlockSpec((blk, PAD), lambda i: (i, 0)),
        out_shape=jax.ShapeDtypeStruct((vocab, PAD), jnp.float32),
    )(table_t, proj_pad)


def _sc_gather(table_pad, idx_flat):
    b_total = idx_flat.shape[0]
    per_w = b_total // NUM_WORKERS
    n_chunks = per_w // CHUNK

    mesh = plsc.VectorSubcoreMesh(core_axis_name="c", subcore_axis_name="s")

    @functools.partial(
        pl.kernel,
        out_type=jax.ShapeDtypeStruct((b_total, PAD), jnp.float32),
        mesh=mesh,
        compiler_params=pltpu.CompilerParams(use_tc_tiling_on_sc=False),
        scratch_types=[
            pltpu.VMEM((per_w,), jnp.int32),
            pltpu.VMEM((CHUNK, PAD), jnp.float32),
            pltpu.VMEM((CHUNK, PAD), jnp.float32),
            pltpu.SemaphoreType.DMA,
            pltpu.SemaphoreType.DMA,
        ],
    )
    def gather_kernel(table_hbm, idx_hbm, out_hbm, idx_v, rows0, rows1, sem0, sem1):
        wid = lax.axis_index("s") * NUM_CORES + lax.axis_index("c")
        base = wid * per_w
        pltpu.sync_copy(idx_hbm.at[pl.ds(base, per_w)], idx_v)

        def start(j, buf, sem):
            pltpu.async_copy(
                table_hbm.at[idx_v.at[pl.ds(j * CHUNK, CHUNK)]], buf, sem)

        def drain(buf, sem):
            pltpu.make_async_copy(
                table_hbm.at[pl.ds(0, CHUNK)], buf, sem).wait()

        n_pairs = n_chunks // 2
        start(0, rows0, sem0)

        def pair_body(p, carry):
            j0 = 2 * p
            start(j0 + 1, rows1, sem1)
            drain(rows0, sem0)
            pltpu.sync_copy(rows0, out_hbm.at[pl.ds(base + j0 * CHUNK, CHUNK)])

            @pl.when(p + 1 < n_pairs)
            def _():
                start(j0 + 2, rows0, sem0)

            drain(rows1, sem1)
            pltpu.sync_copy(
                rows1, out_hbm.at[pl.ds(base + (j0 + 1) * CHUNK, CHUNK)])
            return carry

        lax.fori_loop(0, n_pairs, pair_body, 0)

    return gather_kernel(table_pad, idx_flat)


def _tc_project(low3d, proj_pad):
    seq, bsz, _ = low3d.shape

    lpb = 2

    def mm_kernel(low_ref, w_ref, out_ref):
        for r in range(lpb):
            out_ref[r] = lax.dot_general(
                w_ref[...], low_ref[r],
                (((1,), (1,)), ((), ())),
                preferred_element_type=jnp.float32,
            ) * SCALE

    return pl.pallas_call(
        mm_kernel,
        grid=(seq // lpb,),
        in_specs=[
            pl.BlockSpec((lpb, bsz, PAD), lambda i: (i, 0, 0)),
            pl.BlockSpec((EMB_DIM, PAD), lambda i: (0, 0)),
        ],
        out_specs=pl.BlockSpec((lpb, EMB_DIM, bsz), lambda i: (i, 0, 0)),
        out_shape=jax.ShapeDtypeStruct((seq, EMB_DIM, bsz), jnp.float32),
    )(low3d, proj_pad)


def kernel(x, emb_low, proj_w):
    bsz, seq = x.shape
    table_t = emb_low.T
    idx_flat = x.T.reshape(-1)
    proj_pad = jnp.pad(proj_w, ((0, 0), (0, PAD - RANK)))
    eye_pad = jnp.pad(jnp.eye(RANK, dtype=jnp.float32), ((0, 0), (0, PAD - RANK)))

    table_pad = _tc_repack(table_t, eye_pad)
    low = _sc_gather(table_pad, idx_flat)
    low3d = low.reshape(seq, bsz, PAD)
    out = _tc_project(low3d, proj_pad)
    return out.transpose(2, 0, 1)
